# baseline (speedup 1.0000x reference)
import functools

import jax
import jax.numpy as jnp
from jax import lax
from jax.experimental import pallas as pl
from jax.experimental.pallas import tpu as pltpu
from jax.experimental.pallas import tpu_sc as plsc

N = 10000
E = 320000
D = 128

NC = 2
NS = 16
L = 16
NW = NC * NS
EPT = E // NW
C = 80
NCHUNK = EPT // C
RPT = N // NS

_HIGH = jax.lax.Precision.HIGHEST


def _conv_body(x_hbm, ind_hbm, ea_hbm, zeros_hbm, out_hbm,
               idx_v, xj_v, ea_v, agg_sh, sem, sem_s):
    c = lax.axis_index("c")
    s = lax.axis_index("s")
    wid = c * NS + s

    pltpu.sync_copy(zeros_hbm, agg_sh.at[pl.ds(s * RPT, RPT)])
    plsc.subcore_barrier()

    def issue_loads(j, p):
        pltpu.async_copy(x_hbm.at[idx_v.at[j % 3, 0]], xj_v.at[p], sem)
        pltpu.async_copy(ea_hbm.at[pl.ds(wid * EPT + j * C, C)],
                         ea_v.at[p], sem)

    def wait_loads(j, p):
        pltpu.make_async_copy(x_hbm.at[idx_v.at[0, 0]], xj_v.at[p], sem).wait()
        pltpu.make_async_copy(ea_hbm.at[pl.ds(wid * EPT + j * C, C)],
                              ea_v.at[p], sem).wait()

    def compute(p):
        def row(r, carry2):
            for cc in range(D // L):
                sl = pl.ds(cc * L, L)
                ea_v[p, r, sl] = jnp.maximum(xj_v[p, r, sl] + ea_v[p, r, sl],
                                             0.0)
            return carry2

        lax.fori_loop(0, C, row, 0)

    def wait_scatter():
        pltpu.make_async_copy(ea_v.at[0], agg_sh.at[idx_v.at[0, 1]],
                              sem_s).wait()

    pltpu.sync_copy(ind_hbm.at[wid, 0], idx_v.at[0])
    issue_loads(0, 0)
    pltpu.async_copy(ind_hbm.at[wid, 1], idx_v.at[1], sem)


    def it(j, carry):
        p = j & 1
        wait_loads(j, p)
        pltpu.make_async_copy(ind_hbm.at[wid, j + 1], idx_v.at[0],
                              sem).wait()

        @pl.when(j > 0)
        def _():
            wait_scatter()

        issue_loads(j + 1, 1 - p)

        @pl.when(j + 2 < NCHUNK)
        def _():
            pltpu.async_copy(ind_hbm.at[wid, j + 2], idx_v.at[(j + 2) % 3],
                             sem)

        compute(p)
        pltpu.async_copy(ea_v.at[p], agg_sh.at[idx_v.at[j % 3, 1]], sem_s,
                         add=True)
        return carry

    lax.fori_loop(0, NCHUNK - 1, it, 0)

    wait_loads(NCHUNK - 1, 0)
    wait_scatter()
    compute(0)
    pltpu.sync_copy(ea_v.at[0],
                    agg_sh.at[idx_v.at[(NCHUNK - 1) % 3, 1]], add=True)
    plsc.subcore_barrier()
    base = s * 624
    pltpu.sync_copy(agg_sh.at[pl.ds(base, 624)],
                    out_hbm.at[c, pl.ds(base, 624)])

    @pl.when(s == NS - 1)
    def _tail():
        pltpu.sync_copy(agg_sh.at[pl.ds(16 * 624, N - 16 * 624)],
                        out_hbm.at[c, pl.ds(16 * 624, N - 16 * 624)])


_conv_call = pl.kernel(
    _conv_body,
    out_type=jax.ShapeDtypeStruct((NC, N, D), jnp.float32),
    mesh=plsc.VectorSubcoreMesh(core_axis_name="c", subcore_axis_name="s"),
    scratch_types=[
        pltpu.VMEM((3, 2, C), jnp.int32),
        pltpu.VMEM((2, C, D), jnp.float32),
        pltpu.VMEM((2, C, D), jnp.float32),
        pltpu.VMEM_SHARED((N, D), jnp.float32),
        pltpu.SemaphoreType.DMA,
        pltpu.SemaphoreType.DMA,
    ],
)


BRE = 1000


def _embed_body(z_ref, emb_ref, o_ref):
    zb = z_ref[...]
    oh = (zb == lax.broadcasted_iota(jnp.int32, (BRE, D), 1)).astype(jnp.float32)
    o_ref[...] = jnp.dot(oh, emb_ref[...], preferred_element_type=jnp.float32,
                         precision=_HIGH)


def _embed(z2, emb_pad):
    return pl.pallas_call(
        _embed_body,
        grid=(N // BRE,),
        in_specs=[
            pl.BlockSpec((BRE, 1), lambda i: (i, 0)),
            pl.BlockSpec((D, D), lambda i: (0, 0)),
        ],
        out_specs=pl.BlockSpec((BRE, D), lambda i: (i, 0)),
        out_shape=jax.ShapeDtypeStruct((N, D), jnp.float32),
    )(z2, emb_pad)


BRM = 2000


def _mlp_body(act, p_ref, x_ref, w1_ref, b1_ref, w2_ref, b2_ref, o_ref):
    x = x_ref[...]
    t = p_ref[0] + p_ref[1] + x
    h = jnp.dot(t, w1_ref[...], preferred_element_type=jnp.float32,
                precision=_HIGH) + b1_ref[...]
    h = jnp.maximum(h, 0.0)
    m = jnp.dot(h, w2_ref[...], preferred_element_type=jnp.float32,
                precision=_HIGH) + b2_ref[...]
    if act:
        m = jnp.maximum(m, 0.0)
    o_ref[...] = m + x


def _mlp(parts, x, w1, b1, w2, b2, act):
    return pl.pallas_call(
        functools.partial(_mlp_body, act),
        grid=(N // BRM,),
        in_specs=[
            pl.BlockSpec((NC, BRM, D), lambda i: (0, i, 0)),
            pl.BlockSpec((BRM, D), lambda i: (i, 0)),
            pl.BlockSpec((D, D), lambda i: (0, 0)),
            pl.BlockSpec((1, D), lambda i: (0, 0)),
            pl.BlockSpec((D, D), lambda i: (0, 0)),
            pl.BlockSpec((1, D), lambda i: (0, 0)),
        ],
        out_specs=pl.BlockSpec((BRM, D), lambda i: (i, 0)),
        out_shape=jax.ShapeDtypeStruct((N, D), jnp.float32),
    )(parts, x, w1, b1, w2, b2)


def kernel(z, edge_index, edge_attr, node_emb,
           W1_0, b1_0, W2_0, b2_0,
           W1_1, b1_1, W2_1, b2_1,
           W1_2, b1_2, W2_2, b2_2):
    src3 = edge_index[0].astype(jnp.int32).reshape(NW, NCHUNK, 1, C)
    dst3 = edge_index[1].astype(jnp.int32).reshape(NW, NCHUNK, 1, C)
    ind = jnp.concatenate([src3, dst3], axis=2)
    zeros = jnp.zeros((RPT, D), jnp.float32)
    z2 = z.astype(jnp.int32).reshape(N, 1)
    emb_pad = jnp.zeros((D, D), jnp.float32).at[:100].set(node_emb)

    x = _embed(z2, emb_pad)
    weights = [(W1_0, b1_0, W2_0, b2_0),
               (W1_1, b1_1, W2_1, b2_1),
               (W1_2, b1_2, W2_2, b2_2)]
    for i, (w1, b1, w2, b2) in enumerate(weights):
        parts = _conv_call(x, ind, edge_attr, zeros)
        x = _mlp(parts, x, w1.reshape(D, D), b1.reshape(1, D),
                 w2.reshape(D, D), b2.reshape(1, D), act=(i < 2))
    return x


# speedup vs baseline: 2.3965x; 2.3965x over previous
import functools

import jax
import jax.numpy as jnp
from jax import lax
from jax.experimental import pallas as pl
from jax.experimental.pallas import tpu as pltpu
from jax.experimental.pallas import tpu_sc as plsc

N = 10000
E = 320000
D = 128

NC = 2
NS = 16
L = 16
NW = NC * NS
EPT = E // NW
C = 80
NCHUNK = EPT // C
RPT = N // NS

_HIGH = jax.lax.Precision.HIGHEST


def _conv_body(x_hbm, ind_hbm, ea_hbm, zeros_hbm, out_hbm,
               idx_v, xj_v, ea_v, agg_sh, sem, sem_s):
    c = lax.axis_index("c")
    s = lax.axis_index("s")
    wid = c * NS + s

    pltpu.sync_copy(zeros_hbm, agg_sh.at[pl.ds(s * RPT, RPT)])
    plsc.subcore_barrier()

    def issue_loads(j, k):
        pltpu.async_copy(x_hbm.at[idx_v.at[k % 4, 0]], xj_v.at[k & 1], sem)
        pltpu.async_copy(ea_hbm.at[pl.ds(wid * EPT + j * C, C)],
                         ea_v.at[k & 1], sem)

    def wait_loads(j, k):
        pltpu.make_async_copy(x_hbm.at[idx_v.at[k % 4, 0]], xj_v.at[k & 1],
                              sem).wait()
        pltpu.make_async_copy(ea_hbm.at[pl.ds(wid * EPT + j * C, C)],
                              ea_v.at[k & 1], sem).wait()

    def wait_idx():
        pltpu.make_async_copy(ind_hbm.at[wid, 0], idx_v.at[0], sem).wait()

    def compute(k):
        xp = xj_v.at[k & 1]
        ep = ea_v.at[k & 1]

        def row(r, carry2):
            for cc in range(D // L):
                sl = pl.ds(cc * L, L)
                ep[r, sl] = jnp.maximum(xp[r, sl] + ep[r, sl], 0.0)
            return carry2

        lax.fori_loop(0, C, row, 0)

    def issue_scatter(k, sync=False):
        if sync:
            pltpu.sync_copy(ea_v.at[k & 1], agg_sh.at[idx_v.at[k % 4, 1]],
                            add=True)
        else:
            pltpu.async_copy(ea_v.at[k & 1], agg_sh.at[idx_v.at[k % 4, 1]],
                             sem_s, add=True)

    def wait_scatter():
        pltpu.make_async_copy(ea_v.at[0], agg_sh.at[idx_v.at[0, 1]],
                              sem_s).wait()

    pltpu.sync_copy(ind_hbm.at[wid, 0], idx_v.at[0])
    issue_loads(0, 0)
    pltpu.async_copy(ind_hbm.at[wid, 1], idx_v.at[1], sem)

    def group(g, carry):
        for k in range(4):
            j = g * 4 + k
            wait_loads(j, k)
            wait_idx()
            if k == 0:

                @pl.when(j > 0)
                def _():
                    wait_scatter()
            else:
                wait_scatter()
            issue_loads(j + 1, k + 1)

            @pl.when(j + 2 < NCHUNK)
            def _():
                pltpu.async_copy(ind_hbm.at[wid, j + 2],
                                 idx_v.at[(k + 2) % 4], sem)

            compute(k)
            issue_scatter(k)
        return carry

    lax.fori_loop(0, (NCHUNK - 1) // 4, group, 0)

    wait_loads(NCHUNK - 1, 0)
    wait_scatter()
    compute(0)
    issue_scatter(0, sync=True)
    plsc.subcore_barrier()
    base = s * 624
    pltpu.sync_copy(agg_sh.at[pl.ds(base, 624)],
                    out_hbm.at[c, pl.ds(base, 624)])

    @pl.when(s == NS - 1)
    def _tail():
        pltpu.sync_copy(agg_sh.at[pl.ds(16 * 624, N - 16 * 624)],
                        out_hbm.at[c, pl.ds(16 * 624, N - 16 * 624)])


_conv_call = pl.kernel(
    _conv_body,
    out_type=jax.ShapeDtypeStruct((NC, N, D), jnp.float32),
    mesh=plsc.VectorSubcoreMesh(core_axis_name="c", subcore_axis_name="s"),
    scratch_types=[
        pltpu.VMEM((4, 2, C), jnp.int32),
        pltpu.VMEM((2, C, D), jnp.float32),
        pltpu.VMEM((2, C, D), jnp.float32),
        pltpu.VMEM_SHARED((N, D), jnp.float32),
        pltpu.SemaphoreType.DMA,
        pltpu.SemaphoreType.DMA,
    ],
)


BRE = 1000


def _embed_body(z_ref, emb_ref, o_ref):
    zb = z_ref[...]
    oh = (zb == lax.broadcasted_iota(jnp.int32, (BRE, D), 1)).astype(jnp.float32)
    o_ref[...] = jnp.dot(oh, emb_ref[...], preferred_element_type=jnp.float32,
                         precision=_HIGH)


def _embed(z2, emb_pad):
    return pl.pallas_call(
        _embed_body,
        grid=(N // BRE,),
        in_specs=[
            pl.BlockSpec((BRE, 1), lambda i: (i, 0)),
            pl.BlockSpec((D, D), lambda i: (0, 0)),
        ],
        out_specs=pl.BlockSpec((BRE, D), lambda i: (i, 0)),
        out_shape=jax.ShapeDtypeStruct((N, D), jnp.float32),
    )(z2, emb_pad)


BRM = 2000


def _mlp_body(act, p_ref, x_ref, w1_ref, b1_ref, w2_ref, b2_ref, o_ref):
    x = x_ref[...]
    t = p_ref[0] + p_ref[1] + x
    h = jnp.dot(t, w1_ref[...], preferred_element_type=jnp.float32,
                precision=_HIGH) + b1_ref[...]
    h = jnp.maximum(h, 0.0)
    m = jnp.dot(h, w2_ref[...], preferred_element_type=jnp.float32,
                precision=_HIGH) + b2_ref[...]
    if act:
        m = jnp.maximum(m, 0.0)
    o_ref[...] = m + x


def _mlp(parts, x, w1, b1, w2, b2, act):
    return pl.pallas_call(
        functools.partial(_mlp_body, act),
        grid=(N // BRM,),
        in_specs=[
            pl.BlockSpec((NC, BRM, D), lambda i: (0, i, 0)),
            pl.BlockSpec((BRM, D), lambda i: (i, 0)),
            pl.BlockSpec((D, D), lambda i: (0, 0)),
            pl.BlockSpec((1, D), lambda i: (0, 0)),
            pl.BlockSpec((D, D), lambda i: (0, 0)),
            pl.BlockSpec((1, D), lambda i: (0, 0)),
        ],
        out_specs=pl.BlockSpec((BRM, D), lambda i: (i, 0)),
        out_shape=jax.ShapeDtypeStruct((N, D), jnp.float32),
    )(parts, x, w1, b1, w2, b2)


def kernel(z, edge_index, edge_attr, node_emb,
           W1_0, b1_0, W2_0, b2_0,
           W1_1, b1_1, W2_1, b2_1,
           W1_2, b1_2, W2_2, b2_2):
    src3 = edge_index[0].astype(jnp.int32).reshape(NW, NCHUNK, 1, C)
    dst3 = edge_index[1].astype(jnp.int32).reshape(NW, NCHUNK, 1, C)
    ind = jnp.concatenate([src3, dst3], axis=2)
    zeros = jnp.zeros((RPT, D), jnp.float32)
    z2 = z.astype(jnp.int32).reshape(N, 1)
    emb_pad = jnp.zeros((D, D), jnp.float32).at[:100].set(node_emb)

    x = _embed(z2, emb_pad)
    weights = [(W1_0, b1_0, W2_0, b2_0),
               (W1_1, b1_1, W2_1, b2_1),
               (W1_2, b1_2, W2_2, b2_2)]
    for i, (w1, b1, w2, b2) in enumerate(weights):
        parts = _conv_call(x, ind, edge_attr, zeros)
        x = _mlp(parts, x, w1.reshape(D, D), b1.reshape(1, D),
                 w2.reshape(D, D), b2.reshape(1, D), act=(i < 2))
    return x


# speedup vs baseline: 2.4123x; 1.0066x over previous
import functools

import jax
import jax.numpy as jnp
from jax import lax
from jax.experimental import pallas as pl
from jax.experimental.pallas import tpu as pltpu
from jax.experimental.pallas import tpu_sc as plsc

N = 10000
E = 320000
D = 128

NC = 2
NS = 16
L = 16
NW = NC * NS
EPT = E // NW
C = 80
NCHUNK = EPT // C
RPT = N // NS

_HIGH = jax.lax.Precision.HIGHEST


def _conv_body(x_hbm, ind_hbm, ea_hbm, zeros_hbm, out_hbm,
               idx_v, xj_v, ea_v, agg_sh, sem, sem_s):
    c = lax.axis_index("c")
    s = lax.axis_index("s")
    wid = c * NS + s

    pltpu.sync_copy(zeros_hbm, agg_sh.at[pl.ds(s * RPT, RPT)])
    plsc.subcore_barrier()

    def issue_loads(j, k):
        pltpu.async_copy(x_hbm.at[idx_v.at[k % 4, 0]], xj_v.at[k & 1], sem)
        pltpu.async_copy(ea_hbm.at[pl.ds(wid * EPT + j * C, C)],
                         ea_v.at[k & 1], sem)

    def wait_loads(j, k):
        pltpu.make_async_copy(x_hbm.at[idx_v.at[k % 4, 0]], xj_v.at[k & 1],
                              sem).wait()
        pltpu.make_async_copy(ea_hbm.at[pl.ds(wid * EPT + j * C, C)],
                              ea_v.at[k & 1], sem).wait()

    def wait_idx():
        pltpu.make_async_copy(ind_hbm.at[wid, 0], idx_v.at[0], sem).wait()

    def compute(k):
        xp = xj_v.at[k & 1]
        ep = ea_v.at[k & 1]
        UR = 4

        def row(rr, carry2):
            r0 = rr * UR
            for u in range(UR):
                for cc in range(D // L):
                    sl = pl.ds(cc * L, L)
                    ep[r0 + u, sl] = jnp.maximum(xp[r0 + u, sl] +
                                                 ep[r0 + u, sl], 0.0)
            return carry2

        lax.fori_loop(0, C // UR, row, 0)

    def issue_scatter(k, sync=False):
        if sync:
            pltpu.sync_copy(ea_v.at[k & 1], agg_sh.at[idx_v.at[k % 4, 1]],
                            add=True)
        else:
            pltpu.async_copy(ea_v.at[k & 1], agg_sh.at[idx_v.at[k % 4, 1]],
                             sem_s, add=True)

    def wait_scatter():
        pltpu.make_async_copy(ea_v.at[0], agg_sh.at[idx_v.at[0, 1]],
                              sem_s).wait()

    pltpu.sync_copy(ind_hbm.at[wid, 0], idx_v.at[0])
    issue_loads(0, 0)
    pltpu.async_copy(ind_hbm.at[wid, 1], idx_v.at[1], sem)

    def group(g, carry):
        for k in range(4):
            j = g * 4 + k
            wait_loads(j, k)
            wait_idx()
            if k == 0:

                @pl.when(j > 0)
                def _():
                    wait_scatter()
            else:
                wait_scatter()
            issue_loads(j + 1, k + 1)

            @pl.when(j + 2 < NCHUNK)
            def _():
                pltpu.async_copy(ind_hbm.at[wid, j + 2],
                                 idx_v.at[(k + 2) % 4], sem)

            compute(k)
            issue_scatter(k)
        return carry

    lax.fori_loop(0, (NCHUNK - 1) // 4, group, 0)

    wait_loads(NCHUNK - 1, 0)
    wait_scatter()
    compute(0)
    issue_scatter(0, sync=True)
    plsc.subcore_barrier()
    base = s * 624
    pltpu.sync_copy(agg_sh.at[pl.ds(base, 624)],
                    out_hbm.at[c, pl.ds(base, 624)])

    @pl.when(s == NS - 1)
    def _tail():
        pltpu.sync_copy(agg_sh.at[pl.ds(16 * 624, N - 16 * 624)],
                        out_hbm.at[c, pl.ds(16 * 624, N - 16 * 624)])


_conv_call = pl.kernel(
    _conv_body,
    out_type=jax.ShapeDtypeStruct((NC, N, D), jnp.float32),
    mesh=plsc.VectorSubcoreMesh(core_axis_name="c", subcore_axis_name="s"),
    scratch_types=[
        pltpu.VMEM((4, 2, C), jnp.int32),
        pltpu.VMEM((2, C, D), jnp.float32),
        pltpu.VMEM((2, C, D), jnp.float32),
        pltpu.VMEM_SHARED((N, D), jnp.float32),
        pltpu.SemaphoreType.DMA,
        pltpu.SemaphoreType.DMA,
    ],
)


BRE = 1000


def _embed_body(z_ref, emb_ref, o_ref):
    zb = z_ref[...]
    oh = (zb == lax.broadcasted_iota(jnp.int32, (BRE, D), 1)).astype(jnp.float32)
    o_ref[...] = jnp.dot(oh, emb_ref[...], preferred_element_type=jnp.float32,
                         precision=_HIGH)


def _embed(z2, emb_pad):
    return pl.pallas_call(
        _embed_body,
        grid=(N // BRE,),
        in_specs=[
            pl.BlockSpec((BRE, 1), lambda i: (i, 0)),
            pl.BlockSpec((D, D), lambda i: (0, 0)),
        ],
        out_specs=pl.BlockSpec((BRE, D), lambda i: (i, 0)),
        out_shape=jax.ShapeDtypeStruct((N, D), jnp.float32),
    )(z2, emb_pad)


BRM = 2000


def _mlp_body(act, p_ref, x_ref, w1_ref, b1_ref, w2_ref, b2_ref, o_ref):
    x = x_ref[...]
    t = p_ref[0] + p_ref[1] + x
    h = jnp.dot(t, w1_ref[...], preferred_element_type=jnp.float32,
                precision=_HIGH) + b1_ref[...]
    h = jnp.maximum(h, 0.0)
    m = jnp.dot(h, w2_ref[...], preferred_element_type=jnp.float32,
                precision=_HIGH) + b2_ref[...]
    if act:
        m = jnp.maximum(m, 0.0)
    o_ref[...] = m + x


def _mlp(parts, x, w1, b1, w2, b2, act):
    return pl.pallas_call(
        functools.partial(_mlp_body, act),
        grid=(N // BRM,),
        in_specs=[
            pl.BlockSpec((NC, BRM, D), lambda i: (0, i, 0)),
            pl.BlockSpec((BRM, D), lambda i: (i, 0)),
            pl.BlockSpec((D, D), lambda i: (0, 0)),
            pl.BlockSpec((1, D), lambda i: (0, 0)),
            pl.BlockSpec((D, D), lambda i: (0, 0)),
            pl.BlockSpec((1, D), lambda i: (0, 0)),
        ],
        out_specs=pl.BlockSpec((BRM, D), lambda i: (i, 0)),
        out_shape=jax.ShapeDtypeStruct((N, D), jnp.float32),
    )(parts, x, w1, b1, w2, b2)


def kernel(z, edge_index, edge_attr, node_emb,
           W1_0, b1_0, W2_0, b2_0,
           W1_1, b1_1, W2_1, b2_1,
           W1_2, b1_2, W2_2, b2_2):
    src3 = edge_index[0].astype(jnp.int32).reshape(NW, NCHUNK, 1, C)
    dst3 = edge_index[1].astype(jnp.int32).reshape(NW, NCHUNK, 1, C)
    ind = jnp.concatenate([src3, dst3], axis=2)
    zeros = jnp.zeros((RPT, D), jnp.float32)
    z2 = z.astype(jnp.int32).reshape(N, 1)
    emb_pad = jnp.zeros((D, D), jnp.float32).at[:100].set(node_emb)

    x = _embed(z2, emb_pad)
    weights = [(W1_0, b1_0, W2_0, b2_0),
               (W1_1, b1_1, W2_1, b2_1),
               (W1_2, b1_2, W2_2, b2_2)]
    for i, (w1, b1, w2, b2) in enumerate(weights):
        parts = _conv_call(x, ind, edge_attr, zeros)
        x = _mlp(parts, x, w1.reshape(D, D), b1.reshape(1, D),
                 w2.reshape(D, D), b2.reshape(1, D), act=(i < 2))
    return x
